# baseline (speedup 1.0000x reference)
import functools

import jax
import jax.numpy as jnp
from jax import lax
from jax.experimental import pallas as pl
from jax.experimental.pallas import tpu as pltpu
from jax.experimental.pallas import tpu_sc as plsc

DMODEL = 1024
VOCAB = 100000
BATCH = 4
SEQLEN = 2048
TOTAL = BATCH * SEQLEN
NUM_WORKERS = 32
L_PER_W = SEQLEN // NUM_WORKERS
CHUNK = 32
M_PER_B = L_PER_W // CHUNK
NCHUNKS = BATCH * M_PER_B
NBUF = 3
TC_ROWS = 256


def _position_embedding():
    pos = jnp.arange(SEQLEN, dtype=jnp.float32)[:, None]
    i = jnp.arange(DMODEL, dtype=jnp.float32)[None, :]
    inv_freq = 1.0 / jnp.power(10000.0, 2.0 * i / DMODEL)
    ang = pos * inv_freq
    dim_idx = jnp.arange(DMODEL)
    pe = jnp.where((dim_idx % 2 == 0)[None, :], jnp.sin(ang), jnp.cos(ang))
    return pe.astype(jnp.float32)


@functools.partial(
    pl.kernel,
    out_type=jax.ShapeDtypeStruct((TOTAL, DMODEL), jnp.float32),
    mesh=plsc.VectorSubcoreMesh(core_axis_name="c", subcore_axis_name="s"),
    scratch_types=[
        pltpu.VMEM((BATCH * L_PER_W,), jnp.int32),
    ]
    + [pltpu.VMEM((CHUNK, DMODEL), jnp.float32) for _ in range(NBUF)]
    + [pltpu.SemaphoreType.DMA for _ in range(2 * NBUF)],
)
def _sc_gather(ids_hbm, table_hbm, out_hbm, idx_v, *rest):
    bufs = rest[:NBUF]
    gsems = rest[NBUF:2 * NBUF]
    osems = rest[2 * NBUF:3 * NBUF]

    wid = lax.axis_index("s") * 2 + lax.axis_index("c")
    l0 = wid * L_PER_W

    pltpu.sync_copy(ids_hbm.at[pl.ds(wid * BATCH * L_PER_W, BATCH * L_PER_W)],
                    idx_v)

    def gather(c):
        m, b = c // BATCH, c % BATCH
        return pltpu.async_copy(
            table_hbm.at[idx_v.at[pl.ds(b * L_PER_W + m * CHUNK, CHUNK)]],
            bufs[c % NBUF], gsems[c % NBUF],
        )

    gh = [None] * NBUF
    oh = [None] * NBUF
    for c in range(NBUF):
        gh[c] = gather(c)

    for c in range(NCHUNKS):
        i = c % NBUF
        m, b = c // BATCH, c % BATCH
        gh[i].wait()
        nxt = c - 1 + NBUF
        if c >= 1 and nxt < NCHUNKS:
            j = (c - 1) % NBUF
            oh[j].wait()
            gh[j] = gather(nxt)
        out_base = b * SEQLEN + l0 + m * CHUNK
        oh[i] = pltpu.async_copy(bufs[i], out_hbm.at[pl.ds(out_base, CHUNK)],
                                 osems[i])
    for i in range(NBUF):
        oh[i].wait()


def _tc_add_body(x_ref, pe_ref, o_ref):
    o_ref[...] = x_ref[...] + pe_ref[...]


_tc_add = pl.pallas_call(
    _tc_add_body,
    out_shape=jax.ShapeDtypeStruct((TOTAL, DMODEL), jnp.float32),
    grid=(TOTAL // TC_ROWS,),
    in_specs=[
        pl.BlockSpec((TC_ROWS, DMODEL), lambda i: (i, 0)),
        pl.BlockSpec((TC_ROWS, DMODEL),
                     lambda i: (i % (SEQLEN // TC_ROWS), 0)),
    ],
    out_specs=pl.BlockSpec((TC_ROWS, DMODEL), lambda i: (i, 0)),
)


def kernel(seq, table):
    pe = _position_embedding()
    flat_ids = (seq.astype(jnp.int32)
                .reshape(BATCH, NUM_WORKERS, L_PER_W)
                .transpose(1, 0, 2)
                .reshape(TOTAL))
    gathered = _sc_gather(flat_ids, table)
    out = _tc_add(gathered, pe)
    return out.reshape(BATCH, SEQLEN, DMODEL)


# speedup vs baseline: 1.0635x; 1.0635x over previous
import functools

import numpy as np

import jax
import jax.numpy as jnp
from jax import lax
from jax.experimental import pallas as pl
from jax.experimental.pallas import tpu as pltpu
from jax.experimental.pallas import tpu_sc as plsc

DMODEL = 1024
VOCAB = 100000
BATCH = 4
SEQLEN = 2048
TOTAL = BATCH * SEQLEN
NUM_WORKERS = 32
L_PER_W = SEQLEN // NUM_WORKERS
CHUNK = 32
M_PER_B = L_PER_W // CHUNK
NCHUNKS = BATCH * M_PER_B
NBUF = 3
TC_ROWS = 256


def _position_embedding():
    pos = np.arange(SEQLEN, dtype=np.float64)[:, None]
    i = np.arange(DMODEL, dtype=np.float64)[None, :]
    inv_freq = 1.0 / np.power(10000.0, 2.0 * i / DMODEL)
    ang = pos * inv_freq
    dim_idx = np.arange(DMODEL)
    pe = np.where((dim_idx % 2 == 0)[None, :], np.sin(ang), np.cos(ang))
    return pe.astype(np.float32)


_PE_CONST = _position_embedding()


@functools.partial(
    pl.kernel,
    out_type=jax.ShapeDtypeStruct((TOTAL, DMODEL), jnp.float32),
    mesh=plsc.VectorSubcoreMesh(core_axis_name="c", subcore_axis_name="s"),
    scratch_types=[
        pltpu.VMEM((BATCH * L_PER_W,), jnp.int32),
    ]
    + [pltpu.VMEM((CHUNK, DMODEL), jnp.float32) for _ in range(NBUF)]
    + [pltpu.SemaphoreType.DMA for _ in range(2 * NBUF)],
)
def _sc_gather(ids_hbm, table_hbm, out_hbm, idx_v, *rest):
    bufs = rest[:NBUF]
    gsems = rest[NBUF:2 * NBUF]
    osems = rest[2 * NBUF:3 * NBUF]

    wid = lax.axis_index("s") * 2 + lax.axis_index("c")
    l0 = wid * L_PER_W

    pltpu.sync_copy(ids_hbm.at[pl.ds(wid * BATCH * L_PER_W, BATCH * L_PER_W)],
                    idx_v)

    def gather(c):
        m, b = c // BATCH, c % BATCH
        return pltpu.async_copy(
            table_hbm.at[idx_v.at[pl.ds(b * L_PER_W + m * CHUNK, CHUNK)]],
            bufs[c % NBUF], gsems[c % NBUF],
        )

    gh = [None] * NBUF
    oh = [None] * NBUF
    for c in range(NBUF):
        gh[c] = gather(c)

    for c in range(NCHUNKS):
        i = c % NBUF
        m, b = c // BATCH, c % BATCH
        gh[i].wait()
        nxt = c - 1 + NBUF
        if c >= 1 and nxt < NCHUNKS:
            j = (c - 1) % NBUF
            oh[j].wait()
            gh[j] = gather(nxt)
        out_base = b * SEQLEN + l0 + m * CHUNK
        oh[i] = pltpu.async_copy(bufs[i], out_hbm.at[pl.ds(out_base, CHUNK)],
                                 osems[i])
    for i in range(NBUF):
        oh[i].wait()


def _tc_add_body(x_ref, pe_ref, o_ref):
    o_ref[...] = x_ref[...] + pe_ref[...]


_tc_add = pl.pallas_call(
    _tc_add_body,
    out_shape=jax.ShapeDtypeStruct((TOTAL, DMODEL), jnp.float32),
    grid=(TOTAL // TC_ROWS,),
    in_specs=[
        pl.BlockSpec((TC_ROWS, DMODEL), lambda i: (i, 0)),
        pl.BlockSpec((TC_ROWS, DMODEL),
                     lambda i: (i % (SEQLEN // TC_ROWS), 0)),
    ],
    out_specs=pl.BlockSpec((TC_ROWS, DMODEL), lambda i: (i, 0)),
)


def kernel(seq, table):
    pe = jnp.asarray(_PE_CONST)
    flat_ids = (seq.astype(jnp.int32)
                .reshape(BATCH, NUM_WORKERS, L_PER_W)
                .transpose(1, 0, 2)
                .reshape(TOTAL))
    gathered = _sc_gather(flat_ids, table)
    out = _tc_add(gathered, pe)
    return out.reshape(BATCH, SEQLEN, DMODEL)


# speedup vs baseline: 1.2974x; 1.2199x over previous
import functools

import numpy as np

import jax
import jax.numpy as jnp
from jax import lax
from jax.experimental import pallas as pl
from jax.experimental.pallas import tpu as pltpu
from jax.experimental.pallas import tpu_sc as plsc

DMODEL = 1024
VOCAB = 100000
BATCH = 4
SEQLEN = 2048
TOTAL = BATCH * SEQLEN
NUM_WORKERS = 32
L_PER_W = SEQLEN // NUM_WORKERS
CHUNK = 32
M_PER_B = L_PER_W // CHUNK
NCHUNKS = BATCH * M_PER_B
NBUF = 3
TC_ROWS = 256


def _position_embedding():
    pos = np.arange(SEQLEN, dtype=np.float64)[:, None]
    i = np.arange(DMODEL, dtype=np.float64)[None, :]
    inv_freq = 1.0 / np.power(10000.0, 2.0 * i / DMODEL)
    ang = pos * inv_freq
    dim_idx = np.arange(DMODEL)
    pe = np.where((dim_idx % 2 == 0)[None, :], np.sin(ang), np.cos(ang))
    return pe.astype(np.float32)


_PE_CONST = _position_embedding()


@functools.partial(
    pl.kernel,
    out_type=jax.ShapeDtypeStruct((TOTAL, DMODEL), jnp.float32),
    mesh=plsc.VectorSubcoreMesh(core_axis_name="c", subcore_axis_name="s"),
    scratch_types=[
        pltpu.VMEM((BATCH * L_PER_W,), jnp.int32),
    ]
    + [pltpu.VMEM((CHUNK, DMODEL), jnp.float32) for _ in range(NBUF)]
    + [pltpu.SemaphoreType.DMA for _ in range(2 * NBUF)],
)
def _sc_gather(ids_hbm, table_hbm, out_hbm, idx_v, *rest):
    bufs = rest[:NBUF]
    gsems = rest[NBUF:2 * NBUF]
    osems = rest[2 * NBUF:3 * NBUF]

    wid = lax.axis_index("s") * 2 + lax.axis_index("c")
    l0 = wid * L_PER_W

    pltpu.sync_copy(ids_hbm.at[pl.ds(wid * BATCH * L_PER_W, BATCH * L_PER_W)],
                    idx_v)

    def gather(c):
        m, b = c // BATCH, c % BATCH
        return pltpu.async_copy(
            table_hbm.at[idx_v.at[pl.ds(b * L_PER_W + m * CHUNK, CHUNK)]],
            bufs[c % NBUF], gsems[c % NBUF],
        )

    gh = [None] * NBUF
    oh = [None] * NBUF
    for c in range(NBUF):
        gh[c] = gather(c)

    for c in range(NCHUNKS):
        i = c % NBUF
        m, b = c // BATCH, c % BATCH
        gh[i].wait()
        nxt = c - 1 + NBUF
        if c >= 1 and nxt < NCHUNKS:
            j = (c - 1) % NBUF
            oh[j].wait()
            gh[j] = gather(nxt)
        out_base = b * SEQLEN + l0 + m * CHUNK
        oh[i] = pltpu.async_copy(bufs[i], out_hbm.at[pl.ds(out_base, CHUNK)],
                                 osems[i])
    for i in range(NBUF):
        oh[i].wait()


def _tc_add_body(x_ref, pe_ref, o_ref):
    o_ref[...] = x_ref[...] + pe_ref[...]


_tc_add = pl.pallas_call(
    _tc_add_body,
    out_shape=jax.ShapeDtypeStruct((TOTAL, DMODEL), jnp.float32),
    grid=(BATCH,),
    in_specs=[
        pl.BlockSpec((SEQLEN, DMODEL), lambda i: (i, 0)),
        pl.BlockSpec((SEQLEN, DMODEL), lambda i: (0, 0)),
    ],
    out_specs=pl.BlockSpec((SEQLEN, DMODEL), lambda i: (i, 0)),
)


def kernel(seq, table):
    pe = jnp.asarray(_PE_CONST)
    flat_ids = (seq.astype(jnp.int32)
                .reshape(BATCH, NUM_WORKERS, L_PER_W)
                .transpose(1, 0, 2)
                .reshape(TOTAL))
    gathered = _sc_gather(flat_ids, table)
    out = _tc_add(gathered, pe)
    return out.reshape(BATCH, SEQLEN, DMODEL)
